# Initial kernel scaffold; baseline (speedup 1.0000x reference)
import jax
import jax.numpy as jnp
from jax.experimental import pallas as pl


def kernel(features_1, edge_index_1, features_2, edge_index_2, index, W1, b1, W2, b2, W3, b3, W_att, W_ntn, W_block, b_ntn):
    raise NotImplementedError("write your pallas kernel here")


# speedup vs baseline: 7.5616x; 7.5616x over previous
import functools

import jax
import jax.numpy as jnp
from jax import lax
from jax.experimental import pallas as pl
from jax.experimental.pallas import tpu as pltpu
from jax.experimental.pallas import tpu_sc as plsc

N = 10000
E = 320000
D = 128
F1, F2, F3, K = 128, 64, 32, 16

NW = 32
NCHUNK = 80
CW = 128
EP = NW * NCHUNK * CW
N_PAD = 10112
DUMMY = N
RPT = N_PAD // 16
ZROWS = 8
DEG_W = 16

BLK = 1264
GB = N_PAD // BLK

_HI = lax.Precision.HIGHEST


def _make_agg(F):
    mesh = plsc.VectorSubcoreMesh(core_axis_name="c", subcore_axis_name="s")

    @functools.partial(
        pl.kernel,
        mesh=mesh,
        compiler_params=pltpu.CompilerParams(use_tc_tiling_on_sc=False),
        out_type=jax.ShapeDtypeStruct((4 * N_PAD, F), jnp.float32),
        scratch_types=[
            pltpu.VMEM((NCHUNK, CW), jnp.int32),
            pltpu.VMEM((NCHUNK, CW), jnp.int32),
            pltpu.VMEM((CW, F), jnp.float32),
            pltpu.VMEM((ZROWS, F), jnp.float32),
            pltpu.VMEM_SHARED((N_PAD, F), jnp.float32),
            pltpu.SemaphoreType.DMA,
        ],
    )
    def agg(g_hbm, src_hbm, dst_hbm, out_hbm, srcv, dstv, buf, zbuf, acc, sem):
        cid = lax.axis_index("c")
        sid = lax.axis_index("s")
        wid = sid * 2 + cid

        def zfill(r, carry):
            for c in range(F // 16):
                zbuf[r, pl.ds(c * 16, 16)] = jnp.zeros((16,), jnp.float32)
            return carry
        lax.fori_loop(0, ZROWS, zfill, 0)

        for q in range(2):
            pltpu.sync_copy(src_hbm.at[wid, pl.ds(q * NCHUNK, NCHUNK)], srcv)
            pltpu.sync_copy(dst_hbm.at[wid, pl.ds(q * NCHUNK, NCHUNK)], dstv)

            def zcopy(c, carry):
                pltpu.sync_copy(zbuf, acc.at[pl.ds(sid * RPT + c * ZROWS, ZROWS)])
                return carry
            lax.fori_loop(0, RPT // ZROWS, zcopy, 0)
            plsc.subcore_barrier()

            def chunk(j, carry):
                pltpu.async_copy(g_hbm.at[srcv.at[j]], buf, sem).wait()
                pltpu.sync_copy(buf, acc.at[dstv.at[j]], add=True)
                return carry
            lax.fori_loop(0, NCHUNK, chunk, 0)
            plsc.subcore_barrier()

            base = (q * 2 + cid) * N_PAD + sid * RPT
            pltpu.sync_copy(acc.at[pl.ds(sid * RPT, RPT)],
                            out_hbm.at[pl.ds(base, RPT)])
            plsc.subcore_barrier()

    return agg


_agg_deg = _make_agg(DEG_W)
_agg_f1 = _make_agg(F1)
_agg_f2 = _make_agg(F2)
_agg_f3 = _make_agg(F3)


def _prep1(deg_parts, x, W1):
    def body(deg_ref, x_ref, w_ref, g_ref, dinv_ref):
        degs = deg_ref[0, 0] + deg_ref[0, 1]
        deg = degs[:, 0:1] + 1.0
        dinv = lax.rsqrt(jnp.maximum(deg, 1.0))
        h = jnp.dot(x_ref[0], w_ref[...], preferred_element_type=jnp.float32,
                    precision=_HI)
        g_ref[0] = h * dinv
        dinv_ref[0] = dinv

    return pl.pallas_call(
        body,
        grid=(2, GB),
        in_specs=[
            pl.BlockSpec((1, 2, BLK, DEG_W), lambda q, i: (q, 0, i, 0)),
            pl.BlockSpec((1, BLK, D), lambda q, i: (q, i, 0)),
            pl.BlockSpec((D, F1), lambda q, i: (0, 0)),
        ],
        out_specs=[
            pl.BlockSpec((1, BLK, F1), lambda q, i: (q, i, 0)),
            pl.BlockSpec((1, BLK, 1), lambda q, i: (q, i, 0)),
        ],
        out_shape=[jax.ShapeDtypeStruct((2, N_PAD, F1), jnp.float32),
                   jax.ShapeDtypeStruct((2, N_PAD, 1), jnp.float32)],
    )(deg_parts, x, W1)


def _prep_mid(S, g, dinv, b2d, W, Fin, Fout):
    def body(s_ref, g_ref, d_ref, b_ref, w_ref, o_ref):
        ssum = s_ref[0, 0] + s_ref[0, 1]
        xn = jnp.maximum(d_ref[0] * (ssum + g_ref[0]) + b_ref[...], 0.0)
        o_ref[0] = jnp.dot(xn, w_ref[...], preferred_element_type=jnp.float32,
                           precision=_HI) * d_ref[0]

    return pl.pallas_call(
        body,
        grid=(2, GB),
        in_specs=[
            pl.BlockSpec((1, 2, BLK, Fin), lambda q, i: (q, 0, i, 0)),
            pl.BlockSpec((1, BLK, Fin), lambda q, i: (q, i, 0)),
            pl.BlockSpec((1, BLK, 1), lambda q, i: (q, i, 0)),
            pl.BlockSpec((1, Fin), lambda q, i: (0, 0)),
            pl.BlockSpec((Fin, Fout), lambda q, i: (0, 0)),
        ],
        out_specs=pl.BlockSpec((1, BLK, Fout), lambda q, i: (q, i, 0)),
        out_shape=jax.ShapeDtypeStruct((2, N_PAD, Fout), jnp.float32),
    )(S, g, dinv, b2d, W)


def _final(S3, g3, dinv, b3_2d, W_att, W_ntn, W_blockT, b_ntnT):
    def body(s_ref, g_ref, d_ref, b_ref, watt_ref, wntn_ref, wblkT_ref,
             bntn_ref, o_ref):
        rows = lax.broadcasted_iota(jnp.int32, (N_PAD, 1), 0)
        valid = rows < N
        ps = []
        for q in range(2):
            a = d_ref[q] * (s_ref[q, 0] + s_ref[q, 1] + g_ref[q]) + b_ref[...]
            a = jnp.where(valid, a, 0.0)
            m = jnp.sum(a, axis=0, keepdims=True) / N
            ctx = jnp.tanh(jnp.dot(m, watt_ref[...],
                                   preferred_element_type=jnp.float32,
                                   precision=_HI))
            logits = jnp.sum(a * ctx, axis=1, keepdims=True)
            s = jax.nn.sigmoid(logits)
            s = jnp.where(valid, s, 0.0)
            ps.append(jnp.sum(a * s, axis=0, keepdims=True))
        p1, p2 = ps

        sc = jnp.zeros((1, K), jnp.float32)
        for i in range(F3):
            row = jnp.dot(p2, wntn_ref[i], preferred_element_type=jnp.float32,
                          precision=_HI)
            sc = sc + p1[:, i:i + 1] * row
        comb = jnp.concatenate([p1, p2], axis=1)
        blk = jnp.dot(comb, wblkT_ref[...],
                      preferred_element_type=jnp.float32, precision=_HI)
        o_ref[...] = jnp.maximum(sc + blk + bntn_ref[...], 0.0)

    return pl.pallas_call(
        body,
        out_shape=jax.ShapeDtypeStruct((1, K), jnp.float32),
    )(S3, g3, dinv, b3_2d, W_att, W_ntn, W_blockT, b_ntnT)


def _pack_edges(ei, off):
    src = jnp.concatenate(
        [ei[0].astype(jnp.int32) + off,
         jnp.full((EP - E,), off + DUMMY, jnp.int32)])
    dst = jnp.concatenate(
        [ei[1].astype(jnp.int32),
         jnp.full((EP - E,), DUMMY, jnp.int32)])
    return src.reshape(NW, NCHUNK, CW), dst.reshape(NW, NCHUNK, CW)


def kernel(features_1, edge_index_1, features_2, edge_index_2, index,
           W1, b1, W2, b2, W3, b3, W_att, W_ntn, W_block, b_ntn):
    x = jnp.stack([jnp.pad(features_1, ((0, N_PAD - N), (0, 0))),
                   jnp.pad(features_2, ((0, N_PAD - N), (0, 0)))])

    s1, t1 = _pack_edges(edge_index_1, 0)
    s2, t2 = _pack_edges(edge_index_2, N_PAD)
    src_all = jnp.concatenate([s1, s2], axis=1)
    dst_all = jnp.concatenate([t1, t2], axis=1)

    ones_g = jnp.ones((2 * N_PAD, DEG_W), jnp.float32)
    deg_parts = _agg_deg(ones_g, src_all, dst_all).reshape(2, 2, N_PAD, DEG_W)

    g1, dinv = _prep1(deg_parts, x, W1)
    S1 = _agg_f1(g1.reshape(2 * N_PAD, F1), src_all, dst_all
                 ).reshape(2, 2, N_PAD, F1)
    g2 = _prep_mid(S1, g1, dinv, b1.reshape(1, F1), W2, F1, F2)
    S2 = _agg_f2(g2.reshape(2 * N_PAD, F2), src_all, dst_all
                 ).reshape(2, 2, N_PAD, F2)
    g3 = _prep_mid(S2, g2, dinv, b2.reshape(1, F2), W3, F2, F3)
    S3 = _agg_f3(g3.reshape(2 * N_PAD, F3), src_all, dst_all
                 ).reshape(2, 2, N_PAD, F3)

    return _final(S3, g3, dinv, b3.reshape(1, F3), W_att, W_ntn,
                  W_block.T, b_ntn.reshape(1, K))


# speedup vs baseline: 10.4751x; 1.3853x over previous
import functools

import jax
import jax.numpy as jnp
from jax import lax
from jax.experimental import pallas as pl
from jax.experimental.pallas import tpu as pltpu
from jax.experimental.pallas import tpu_sc as plsc

N = 10000
E = 320000
D = 128
F1, F2, F3, K = 128, 64, 32, 16

NW = 32
NCHUNK = 80
CW = 128
EP = NW * NCHUNK * CW
N_PAD = 10112
DUMMY = N
RPT = N_PAD // 16
ZROWS = 8
DEG_W = 16

BLK = 1264
GB = N_PAD // BLK

_HI = lax.Precision.HIGHEST


NH = NCHUNK // 2


def _make_agg(F, nbuf):
    mesh = plsc.VectorSubcoreMesh(core_axis_name="c", subcore_axis_name="s")

    @functools.partial(
        pl.kernel,
        mesh=mesh,
        compiler_params=pltpu.CompilerParams(use_tc_tiling_on_sc=False),
        out_type=jax.ShapeDtypeStruct((4 * N_PAD, F), jnp.float32),
        scratch_types=[
            pltpu.VMEM((NH, CW), jnp.int32),
            pltpu.VMEM((NH, CW), jnp.int32),
            [pltpu.VMEM((CW, F), jnp.float32)] * nbuf,
            pltpu.VMEM((ZROWS, F), jnp.float32),
            pltpu.VMEM_SHARED((N_PAD, F), jnp.float32),
            [pltpu.SemaphoreType.DMA] * nbuf,
        ],
    )
    def agg(g_hbm, src_hbm, dst_hbm, out_hbm, srcv, dstv, bufs, zbuf, acc,
            sems):
        cid = lax.axis_index("c")
        sid = lax.axis_index("s")
        wid = sid * 2 + cid

        def zfill(r, carry):
            for c in range(F // 16):
                zbuf[r, pl.ds(c * 16, 16)] = jnp.zeros((16,), jnp.float32)
            return carry
        lax.fori_loop(0, ZROWS, zfill, 0)

        for q in range(2):
            def zcopy(c, carry):
                pltpu.sync_copy(zbuf, acc.at[pl.ds(sid * RPT + c * ZROWS, ZROWS)])
                return carry
            lax.fori_loop(0, RPT // ZROWS, zcopy, 0)
            plsc.subcore_barrier()

            for h in range(2):
                base_c = q * NCHUNK + h * NH
                pltpu.sync_copy(src_hbm.at[wid, pl.ds(base_c, NH)], srcv)
                pltpu.sync_copy(dst_hbm.at[wid, pl.ds(base_c, NH)], dstv)

                for b in range(nbuf):
                    pltpu.async_copy(g_hbm.at[srcv.at[b]], bufs[b], sems[b])

                def grp(i, carry):
                    for b in range(nbuf):
                        j = i * nbuf + b
                        pltpu.make_async_copy(
                            g_hbm.at[srcv.at[j]], bufs[b], sems[b]).wait()
                        pltpu.sync_copy(bufs[b], acc.at[dstv.at[j]], add=True)

                        @pl.when(j + nbuf < NH)
                        def _fire():
                            pltpu.async_copy(
                                g_hbm.at[srcv.at[j + nbuf]], bufs[b], sems[b])
                    return carry
                lax.fori_loop(0, NH // nbuf, grp, 0)
            plsc.subcore_barrier()

            base = (q * 2 + cid) * N_PAD + sid * RPT
            pltpu.sync_copy(acc.at[pl.ds(sid * RPT, RPT)],
                            out_hbm.at[pl.ds(base, RPT)])
            plsc.subcore_barrier()

    return agg


def _make_deg():
    mesh = plsc.VectorSubcoreMesh(core_axis_name="c", subcore_axis_name="s")
    GRP = 8

    @functools.partial(
        pl.kernel,
        mesh=mesh,
        compiler_params=pltpu.CompilerParams(use_tc_tiling_on_sc=False),
        out_type=jax.ShapeDtypeStruct((4 * N_PAD, DEG_W), jnp.float32),
        scratch_types=[
            pltpu.VMEM((NCHUNK, CW), jnp.int32),
            pltpu.VMEM((CW, DEG_W), jnp.float32),
            pltpu.VMEM((ZROWS, DEG_W), jnp.float32),
            pltpu.VMEM_SHARED((N_PAD, DEG_W), jnp.float32),
            pltpu.SemaphoreType.DMA,
        ],
    )
    def deg(dst_hbm, out_hbm, dstv, ones, zbuf, acc, sem):
        cid = lax.axis_index("c")
        sid = lax.axis_index("s")
        wid = sid * 2 + cid

        def fill(r, carry):
            zbuf[r, pl.ds(0, 16)] = jnp.zeros((16,), jnp.float32)
            return carry
        lax.fori_loop(0, ZROWS, fill, 0)

        def ofill(r, carry):
            ones[r, pl.ds(0, 16)] = jnp.ones((16,), jnp.float32)
            return carry
        lax.fori_loop(0, CW, ofill, 0)

        for q in range(2):
            pltpu.sync_copy(dst_hbm.at[wid, pl.ds(q * NCHUNK, NCHUNK)], dstv)

            def zcopy(c, carry):
                pltpu.sync_copy(zbuf, acc.at[pl.ds(sid * RPT + c * ZROWS, ZROWS)])
                return carry
            lax.fori_loop(0, RPT // ZROWS, zcopy, 0)
            plsc.subcore_barrier()

            def grp(i, carry):
                for b in range(GRP):
                    pltpu.async_copy(ones, acc.at[dstv.at[i * GRP + b]], sem,
                                     add=True)
                for b in range(GRP):
                    pltpu.make_async_copy(
                        ones, acc.at[dstv.at[i * GRP + b]], sem).wait()
                return carry
            lax.fori_loop(0, NCHUNK // GRP, grp, 0)
            plsc.subcore_barrier()

            base = (q * 2 + cid) * N_PAD + sid * RPT
            pltpu.sync_copy(acc.at[pl.ds(sid * RPT, RPT)],
                            out_hbm.at[pl.ds(base, RPT)])
            plsc.subcore_barrier()

    return deg


_agg_deg = _make_deg()
_agg_f1 = _make_agg(F1, 2)
_agg_f2 = _make_agg(F2, 4)
_agg_f3 = _make_agg(F3, 4)


def _prep1(deg_parts, x, W1):
    def body(deg_ref, x_ref, w_ref, g_ref, dinv_ref):
        degs = deg_ref[0, 0] + deg_ref[0, 1]
        deg = degs[:, 0:1] + 1.0
        dinv = lax.rsqrt(jnp.maximum(deg, 1.0))
        h = jnp.dot(x_ref[0], w_ref[...], preferred_element_type=jnp.float32,
                    precision=_HI)
        g_ref[0] = h * dinv
        dinv_ref[0] = dinv

    return pl.pallas_call(
        body,
        grid=(2, GB),
        in_specs=[
            pl.BlockSpec((1, 2, BLK, DEG_W), lambda q, i: (q, 0, i, 0)),
            pl.BlockSpec((1, BLK, D), lambda q, i: (q, i, 0)),
            pl.BlockSpec((D, F1), lambda q, i: (0, 0)),
        ],
        out_specs=[
            pl.BlockSpec((1, BLK, F1), lambda q, i: (q, i, 0)),
            pl.BlockSpec((1, BLK, 1), lambda q, i: (q, i, 0)),
        ],
        out_shape=[jax.ShapeDtypeStruct((2, N_PAD, F1), jnp.float32),
                   jax.ShapeDtypeStruct((2, N_PAD, 1), jnp.float32)],
    )(deg_parts, x, W1)


def _prep_mid(S, g, dinv, b2d, W, Fin, Fout):
    def body(s_ref, g_ref, d_ref, b_ref, w_ref, o_ref):
        ssum = s_ref[0, 0] + s_ref[0, 1]
        xn = jnp.maximum(d_ref[0] * (ssum + g_ref[0]) + b_ref[...], 0.0)
        o_ref[0] = jnp.dot(xn, w_ref[...], preferred_element_type=jnp.float32,
                           precision=_HI) * d_ref[0]

    return pl.pallas_call(
        body,
        grid=(2, GB),
        in_specs=[
            pl.BlockSpec((1, 2, BLK, Fin), lambda q, i: (q, 0, i, 0)),
            pl.BlockSpec((1, BLK, Fin), lambda q, i: (q, i, 0)),
            pl.BlockSpec((1, BLK, 1), lambda q, i: (q, i, 0)),
            pl.BlockSpec((1, Fin), lambda q, i: (0, 0)),
            pl.BlockSpec((Fin, Fout), lambda q, i: (0, 0)),
        ],
        out_specs=pl.BlockSpec((1, BLK, Fout), lambda q, i: (q, i, 0)),
        out_shape=jax.ShapeDtypeStruct((2, N_PAD, Fout), jnp.float32),
    )(S, g, dinv, b2d, W)


def _final(S3, g3, dinv, b3_2d, W_att, W_ntn, W_blockT, b_ntnT):
    def body(s_ref, g_ref, d_ref, b_ref, watt_ref, wntn_ref, wblkT_ref,
             bntn_ref, o_ref):
        rows = lax.broadcasted_iota(jnp.int32, (N_PAD, 1), 0)
        valid = rows < N
        ps = []
        for q in range(2):
            a = d_ref[q] * (s_ref[q, 0] + s_ref[q, 1] + g_ref[q]) + b_ref[...]
            a = jnp.where(valid, a, 0.0)
            m = jnp.sum(a, axis=0, keepdims=True) / N
            ctx = jnp.tanh(jnp.dot(m, watt_ref[...],
                                   preferred_element_type=jnp.float32,
                                   precision=_HI))
            logits = jnp.sum(a * ctx, axis=1, keepdims=True)
            s = jax.nn.sigmoid(logits)
            s = jnp.where(valid, s, 0.0)
            ps.append(jnp.sum(a * s, axis=0, keepdims=True))
        p1, p2 = ps

        sc = jnp.zeros((1, K), jnp.float32)
        for i in range(F3):
            row = jnp.dot(p2, wntn_ref[i], preferred_element_type=jnp.float32,
                          precision=_HI)
            sc = sc + p1[:, i:i + 1] * row
        comb = jnp.concatenate([p1, p2], axis=1)
        blk = jnp.dot(comb, wblkT_ref[...],
                      preferred_element_type=jnp.float32, precision=_HI)
        o_ref[...] = jnp.maximum(sc + blk + bntn_ref[...], 0.0)

    return pl.pallas_call(
        body,
        out_shape=jax.ShapeDtypeStruct((1, K), jnp.float32),
    )(S3, g3, dinv, b3_2d, W_att, W_ntn, W_blockT, b_ntnT)


def _pack_edges(ei, off):
    src = jnp.concatenate(
        [ei[0].astype(jnp.int32) + off,
         jnp.full((EP - E,), off + DUMMY, jnp.int32)])
    dst = jnp.concatenate(
        [ei[1].astype(jnp.int32),
         jnp.full((EP - E,), DUMMY, jnp.int32)])
    return src.reshape(NW, NCHUNK, CW), dst.reshape(NW, NCHUNK, CW)


def kernel(features_1, edge_index_1, features_2, edge_index_2, index,
           W1, b1, W2, b2, W3, b3, W_att, W_ntn, W_block, b_ntn):
    x = jnp.stack([jnp.pad(features_1, ((0, N_PAD - N), (0, 0))),
                   jnp.pad(features_2, ((0, N_PAD - N), (0, 0)))])

    s1, t1 = _pack_edges(edge_index_1, 0)
    s2, t2 = _pack_edges(edge_index_2, N_PAD)
    src_all = jnp.concatenate([s1, s2], axis=1)
    dst_all = jnp.concatenate([t1, t2], axis=1)

    deg_parts = _agg_deg(dst_all).reshape(2, 2, N_PAD, DEG_W)

    g1, dinv = _prep1(deg_parts, x, W1)
    S1 = _agg_f1(g1.reshape(2 * N_PAD, F1), src_all, dst_all
                 ).reshape(2, 2, N_PAD, F1)
    g2 = _prep_mid(S1, g1, dinv, b1.reshape(1, F1), W2, F1, F2)
    S2 = _agg_f2(g2.reshape(2 * N_PAD, F2), src_all, dst_all
                 ).reshape(2, 2, N_PAD, F2)
    g3 = _prep_mid(S2, g2, dinv, b2.reshape(1, F2), W3, F2, F3)
    S3 = _agg_f3(g3.reshape(2 * N_PAD, F3), src_all, dst_all
                 ).reshape(2, 2, N_PAD, F3)

    return _final(S3, g3, dinv, b3.reshape(1, F3), W_att, W_ntn,
                  W_block.T, b_ntn.reshape(1, K))


# speedup vs baseline: 11.2661x; 1.0755x over previous
import functools

import jax
import jax.numpy as jnp
from jax import lax
from jax.experimental import pallas as pl
from jax.experimental.pallas import tpu as pltpu
from jax.experimental.pallas import tpu_sc as plsc

N = 10000
E = 320000
D = 128
F1, F2, F3, K = 128, 64, 32, 16

NW = 32
NCHUNK = 80
CW = 128
EP = NW * NCHUNK * CW
NCG = 2 * 16 * NCHUNK
NC0 = 120
NC1 = 160 - NC0
BCH = 20
N_PAD = 10112
DUMMY = N
RPT = N_PAD // 16
ZROWS = 8
DEG_W = 16

BLK = 1264
GB = N_PAD // BLK

_HI = lax.Precision.HIGHEST


def _make_agg(F, nbuf):
    mesh = plsc.VectorSubcoreMesh(core_axis_name="c", subcore_axis_name="s")

    @functools.partial(
        pl.kernel,
        mesh=mesh,
        compiler_params=pltpu.CompilerParams(use_tc_tiling_on_sc=False),
        out_type=jax.ShapeDtypeStruct((4 * N_PAD, F), jnp.float32),
        scratch_types=[
            pltpu.VMEM((BCH, CW), jnp.int32),
            pltpu.VMEM((BCH, CW), jnp.int32),
            [pltpu.VMEM((CW, F), jnp.float32)] * nbuf,
            pltpu.VMEM((ZROWS, F), jnp.float32),
            pltpu.VMEM_SHARED((N_PAD, F), jnp.float32),
            [pltpu.SemaphoreType.DMA] * nbuf,
        ],
    )
    def agg(g_hbm, src_hbm, dst_hbm, out_hbm, srcv, dstv, bufs, zbuf, acc,
            sems):
        cid = lax.axis_index("c")
        sid = lax.axis_index("s")
        coff = sid * (NC0 + NC1) + cid * NC0
        nblk = jnp.where(cid == 0, NC0 // BCH, NC1 // BCH)

        def zfill(r, carry):
            for c in range(F // 16):
                zbuf[r, pl.ds(c * 16, 16)] = jnp.zeros((16,), jnp.float32)
            return carry
        lax.fori_loop(0, ZROWS, zfill, 0)

        for q in range(2):
            def zcopy(c, carry):
                pltpu.sync_copy(zbuf, acc.at[pl.ds(sid * RPT + c * ZROWS, ZROWS)])
                return carry
            lax.fori_loop(0, RPT // ZROWS, zcopy, 0)
            plsc.subcore_barrier()

            def block(ib, carry):
                row0 = q * NCG + coff + ib * BCH
                pltpu.sync_copy(src_hbm.at[pl.ds(row0, BCH)], srcv)
                pltpu.sync_copy(dst_hbm.at[pl.ds(row0, BCH)], dstv)

                for b in range(nbuf):
                    pltpu.async_copy(g_hbm.at[srcv.at[b]], bufs[b], sems[b])

                def grp(i, carry2):
                    for b in range(nbuf):
                        j = i * nbuf + b
                        pltpu.make_async_copy(
                            g_hbm.at[srcv.at[j]], bufs[b], sems[b]).wait()
                        pltpu.sync_copy(bufs[b], acc.at[dstv.at[j]], add=True)

                        @pl.when(j + nbuf < BCH)
                        def _fire():
                            pltpu.async_copy(
                                g_hbm.at[srcv.at[j + nbuf]], bufs[b], sems[b])
                    return carry2
                lax.fori_loop(0, BCH // nbuf, grp, 0)
                return carry
            lax.fori_loop(0, nblk, block, 0)
            plsc.subcore_barrier()

            base = (q * 2 + cid) * N_PAD + sid * RPT
            pltpu.sync_copy(acc.at[pl.ds(sid * RPT, RPT)],
                            out_hbm.at[pl.ds(base, RPT)])
            plsc.subcore_barrier()

    return agg


def _make_deg():
    mesh = plsc.VectorSubcoreMesh(core_axis_name="c", subcore_axis_name="s")
    GRP = 10

    @functools.partial(
        pl.kernel,
        mesh=mesh,
        compiler_params=pltpu.CompilerParams(use_tc_tiling_on_sc=False),
        out_type=jax.ShapeDtypeStruct((4 * N_PAD, DEG_W), jnp.float32),
        scratch_types=[
            pltpu.VMEM((BCH, CW), jnp.int32),
            pltpu.VMEM((CW, DEG_W), jnp.float32),
            pltpu.VMEM((ZROWS, DEG_W), jnp.float32),
            pltpu.VMEM_SHARED((N_PAD, DEG_W), jnp.float32),
            pltpu.SemaphoreType.DMA,
        ],
    )
    def deg(dst_hbm, out_hbm, dstv, ones, zbuf, acc, sem):
        cid = lax.axis_index("c")
        sid = lax.axis_index("s")
        coff = sid * (NC0 + NC1) + cid * NC0
        nblk = jnp.where(cid == 0, NC0 // BCH, NC1 // BCH)

        def fill(r, carry):
            zbuf[r, pl.ds(0, 16)] = jnp.zeros((16,), jnp.float32)
            return carry
        lax.fori_loop(0, ZROWS, fill, 0)

        def ofill(r, carry):
            ones[r, pl.ds(0, 16)] = jnp.ones((16,), jnp.float32)
            return carry
        lax.fori_loop(0, CW, ofill, 0)

        for q in range(2):
            def zcopy(c, carry):
                pltpu.sync_copy(zbuf, acc.at[pl.ds(sid * RPT + c * ZROWS, ZROWS)])
                return carry
            lax.fori_loop(0, RPT // ZROWS, zcopy, 0)
            plsc.subcore_barrier()

            def block(ib, carry):
                row0 = q * NCG + coff + ib * BCH
                pltpu.sync_copy(dst_hbm.at[pl.ds(row0, BCH)], dstv)

                def grp(i, carry2):
                    for b in range(GRP):
                        pltpu.async_copy(ones, acc.at[dstv.at[i * GRP + b]],
                                         sem, add=True)
                    for b in range(GRP):
                        pltpu.make_async_copy(
                            ones, acc.at[dstv.at[i * GRP + b]], sem).wait()
                    return carry2
                lax.fori_loop(0, BCH // GRP, grp, 0)
                return carry
            lax.fori_loop(0, nblk, block, 0)
            plsc.subcore_barrier()

            base = (q * 2 + cid) * N_PAD + sid * RPT
            pltpu.sync_copy(acc.at[pl.ds(sid * RPT, RPT)],
                            out_hbm.at[pl.ds(base, RPT)])
            plsc.subcore_barrier()

    return deg


_agg_deg = _make_deg()
_agg_f1 = _make_agg(F1, 2)
_agg_f2 = _make_agg(F2, 4)
_agg_f3 = _make_agg(F3, 4)


def _prep1(deg_parts, x, W1):
    def body(deg_ref, x_ref, w_ref, g_ref, dinv_ref):
        degs = deg_ref[0, 0] + deg_ref[0, 1]
        deg = degs[:, 0:1] + 1.0
        dinv = lax.rsqrt(jnp.maximum(deg, 1.0))
        h = jnp.dot(x_ref[0], w_ref[...], preferred_element_type=jnp.float32,
                    precision=_HI)
        g_ref[0] = h * dinv
        dinv_ref[0] = dinv

    return pl.pallas_call(
        body,
        grid=(2, GB),
        in_specs=[
            pl.BlockSpec((1, 2, BLK, DEG_W), lambda q, i: (q, 0, i, 0)),
            pl.BlockSpec((1, BLK, D), lambda q, i: (q, i, 0)),
            pl.BlockSpec((D, F1), lambda q, i: (0, 0)),
        ],
        out_specs=[
            pl.BlockSpec((1, BLK, F1), lambda q, i: (q, i, 0)),
            pl.BlockSpec((1, BLK, 1), lambda q, i: (q, i, 0)),
        ],
        out_shape=[jax.ShapeDtypeStruct((2, N_PAD, F1), jnp.float32),
                   jax.ShapeDtypeStruct((2, N_PAD, 1), jnp.float32)],
    )(deg_parts, x, W1)


def _prep_mid(S, g, dinv, b2d, W, Fin, Fout):
    def body(s_ref, g_ref, d_ref, b_ref, w_ref, o_ref):
        ssum = s_ref[0, 0] + s_ref[0, 1]
        xn = jnp.maximum(d_ref[0] * (ssum + g_ref[0]) + b_ref[...], 0.0)
        o_ref[0] = jnp.dot(xn, w_ref[...], preferred_element_type=jnp.float32,
                           precision=_HI) * d_ref[0]

    return pl.pallas_call(
        body,
        grid=(2, GB),
        in_specs=[
            pl.BlockSpec((1, 2, BLK, Fin), lambda q, i: (q, 0, i, 0)),
            pl.BlockSpec((1, BLK, Fin), lambda q, i: (q, i, 0)),
            pl.BlockSpec((1, BLK, 1), lambda q, i: (q, i, 0)),
            pl.BlockSpec((1, Fin), lambda q, i: (0, 0)),
            pl.BlockSpec((Fin, Fout), lambda q, i: (0, 0)),
        ],
        out_specs=pl.BlockSpec((1, BLK, Fout), lambda q, i: (q, i, 0)),
        out_shape=jax.ShapeDtypeStruct((2, N_PAD, Fout), jnp.float32),
    )(S, g, dinv, b2d, W)


def _final(S3, g3, dinv, b3_2d, W_att, W_ntn, W_blockT, b_ntnT):
    def body(s_ref, g_ref, d_ref, b_ref, watt_ref, wntn_ref, wblkT_ref,
             bntn_ref, o_ref):
        rows = lax.broadcasted_iota(jnp.int32, (N_PAD, 1), 0)
        valid = rows < N
        ps = []
        for q in range(2):
            a = d_ref[q] * (s_ref[q, 0] + s_ref[q, 1] + g_ref[q]) + b_ref[...]
            a = jnp.where(valid, a, 0.0)
            m = jnp.sum(a, axis=0, keepdims=True) / N
            ctx = jnp.tanh(jnp.dot(m, watt_ref[...],
                                   preferred_element_type=jnp.float32,
                                   precision=_HI))
            logits = jnp.sum(a * ctx, axis=1, keepdims=True)
            s = jax.nn.sigmoid(logits)
            s = jnp.where(valid, s, 0.0)
            ps.append(jnp.sum(a * s, axis=0, keepdims=True))
        p1, p2 = ps

        sc = jnp.zeros((1, K), jnp.float32)
        for i in range(F3):
            row = jnp.dot(p2, wntn_ref[i], preferred_element_type=jnp.float32,
                          precision=_HI)
            sc = sc + p1[:, i:i + 1] * row
        comb = jnp.concatenate([p1, p2], axis=1)
        blk = jnp.dot(comb, wblkT_ref[...],
                      preferred_element_type=jnp.float32, precision=_HI)
        o_ref[...] = jnp.maximum(sc + blk + bntn_ref[...], 0.0)

    return pl.pallas_call(
        body,
        out_shape=jax.ShapeDtypeStruct((1, K), jnp.float32),
    )(S3, g3, dinv, b3_2d, W_att, W_ntn, W_blockT, b_ntnT)


def _pack_edges(ei, off):
    src = jnp.concatenate(
        [ei[0].astype(jnp.int32) + off,
         jnp.full((EP - E,), off + DUMMY, jnp.int32)])
    dst = jnp.concatenate(
        [ei[1].astype(jnp.int32),
         jnp.full((EP - E,), DUMMY, jnp.int32)])
    return src.reshape(NCG, CW), dst.reshape(NCG, CW)


def kernel(features_1, edge_index_1, features_2, edge_index_2, index,
           W1, b1, W2, b2, W3, b3, W_att, W_ntn, W_block, b_ntn):
    x = jnp.stack([jnp.pad(features_1, ((0, N_PAD - N), (0, 0))),
                   jnp.pad(features_2, ((0, N_PAD - N), (0, 0)))])

    s1, t1 = _pack_edges(edge_index_1, 0)
    s2, t2 = _pack_edges(edge_index_2, N_PAD)
    src_all = jnp.concatenate([s1, s2], axis=0)
    dst_all = jnp.concatenate([t1, t2], axis=0)

    deg_parts = _agg_deg(dst_all).reshape(2, 2, N_PAD, DEG_W)

    g1, dinv = _prep1(deg_parts, x, W1)
    S1 = _agg_f1(g1.reshape(2 * N_PAD, F1), src_all, dst_all
                 ).reshape(2, 2, N_PAD, F1)
    g2 = _prep_mid(S1, g1, dinv, b1.reshape(1, F1), W2, F1, F2)
    S2 = _agg_f2(g2.reshape(2 * N_PAD, F2), src_all, dst_all
                 ).reshape(2, 2, N_PAD, F2)
    g3 = _prep_mid(S2, g2, dinv, b2.reshape(1, F2), W3, F2, F3)
    S3 = _agg_f3(g3.reshape(2 * N_PAD, F3), src_all, dst_all
                 ).reshape(2, 2, N_PAD, F3)

    return _final(S3, g3, dinv, b3.reshape(1, F3), W_att, W_ntn,
                  W_block.T, b_ntn.reshape(1, K))


# speedup vs baseline: 11.3611x; 1.0084x over previous
import functools

import jax
import jax.numpy as jnp
from jax import lax
from jax.experimental import pallas as pl
from jax.experimental.pallas import tpu as pltpu
from jax.experimental.pallas import tpu_sc as plsc

N = 10000
E = 320000
D = 128
F1, F2, F3, K = 128, 64, 32, 16

NW = 32
NCHUNK = 80
CW = 128
EP = NW * NCHUNK * CW
NCG = 2 * 16 * NCHUNK
NC0 = 120
NC1 = 160 - NC0
BCH = 40
N_PAD = 10112
DUMMY = N
RPT = N_PAD // 16
ZROWS = 8
DEG_W = 16

BLK = 1264
GB = N_PAD // BLK

_HI = lax.Precision.HIGHEST


def _make_agg(F, nbuf):
    mesh = plsc.VectorSubcoreMesh(core_axis_name="c", subcore_axis_name="s")

    @functools.partial(
        pl.kernel,
        mesh=mesh,
        compiler_params=pltpu.CompilerParams(use_tc_tiling_on_sc=False),
        out_type=jax.ShapeDtypeStruct((4 * N_PAD, F), jnp.float32),
        scratch_types=[
            pltpu.VMEM((BCH, CW), jnp.int32),
            pltpu.VMEM((BCH, CW), jnp.int32),
            [pltpu.VMEM((CW, F), jnp.float32)] * nbuf,
            pltpu.VMEM((ZROWS, F), jnp.float32),
            pltpu.VMEM_SHARED((N_PAD, F), jnp.float32),
            [pltpu.SemaphoreType.DMA] * nbuf,
        ],
    )
    def agg(g_hbm, src_hbm, dst_hbm, out_hbm, srcv, dstv, bufs, zbuf, acc,
            sems):
        cid = lax.axis_index("c")
        sid = lax.axis_index("s")
        coff = sid * (NC0 + NC1) + cid * NC0
        nblk = jnp.where(cid == 0, NC0 // BCH, NC1 // BCH)

        def zfill(r, carry):
            for c in range(F // 16):
                zbuf[r, pl.ds(c * 16, 16)] = jnp.zeros((16,), jnp.float32)
            return carry
        lax.fori_loop(0, ZROWS, zfill, 0)

        for q in range(2):
            def zcopy(c, carry):
                pltpu.sync_copy(zbuf, acc.at[pl.ds(sid * RPT + c * ZROWS, ZROWS)])
                return carry
            lax.fori_loop(0, RPT // ZROWS, zcopy, 0)
            plsc.subcore_barrier()

            def block(ib, carry):
                row0 = q * NCG + coff + ib * BCH
                pltpu.sync_copy(src_hbm.at[pl.ds(row0, BCH)], srcv)
                pltpu.sync_copy(dst_hbm.at[pl.ds(row0, BCH)], dstv)

                for b in range(nbuf):
                    pltpu.async_copy(g_hbm.at[srcv.at[b]], bufs[b], sems[b])

                def grp(i, carry2):
                    for b in range(nbuf):
                        j = i * nbuf + b
                        pltpu.make_async_copy(
                            g_hbm.at[srcv.at[j]], bufs[b], sems[b]).wait()
                        pltpu.sync_copy(bufs[b], acc.at[dstv.at[j]], add=True)

                        @pl.when(j + nbuf < BCH)
                        def _fire():
                            pltpu.async_copy(
                                g_hbm.at[srcv.at[j + nbuf]], bufs[b], sems[b])
                    return carry2
                lax.fori_loop(0, BCH // nbuf, grp, 0)
                return carry
            lax.fori_loop(0, nblk, block, 0)
            plsc.subcore_barrier()

            base = (q * 2 + cid) * N_PAD + sid * RPT
            pltpu.sync_copy(acc.at[pl.ds(sid * RPT, RPT)],
                            out_hbm.at[pl.ds(base, RPT)])
            plsc.subcore_barrier()

    return agg


def _make_deg():
    mesh = plsc.VectorSubcoreMesh(core_axis_name="c", subcore_axis_name="s")
    GRP = 10

    @functools.partial(
        pl.kernel,
        mesh=mesh,
        compiler_params=pltpu.CompilerParams(use_tc_tiling_on_sc=False),
        out_type=jax.ShapeDtypeStruct((4 * N_PAD, DEG_W), jnp.float32),
        scratch_types=[
            pltpu.VMEM((BCH, CW), jnp.int32),
            pltpu.VMEM((CW, DEG_W), jnp.float32),
            pltpu.VMEM((ZROWS, DEG_W), jnp.float32),
            pltpu.VMEM_SHARED((N_PAD, DEG_W), jnp.float32),
            pltpu.SemaphoreType.DMA,
        ],
    )
    def deg(dst_hbm, out_hbm, dstv, ones, zbuf, acc, sem):
        cid = lax.axis_index("c")
        sid = lax.axis_index("s")
        coff = sid * (NC0 + NC1) + cid * NC0
        nblk = jnp.where(cid == 0, NC0 // BCH, NC1 // BCH)

        def fill(r, carry):
            zbuf[r, pl.ds(0, 16)] = jnp.zeros((16,), jnp.float32)
            return carry
        lax.fori_loop(0, ZROWS, fill, 0)

        def ofill(r, carry):
            ones[r, pl.ds(0, 16)] = jnp.ones((16,), jnp.float32)
            return carry
        lax.fori_loop(0, CW, ofill, 0)

        for q in range(2):
            def zcopy(c, carry):
                pltpu.sync_copy(zbuf, acc.at[pl.ds(sid * RPT + c * ZROWS, ZROWS)])
                return carry
            lax.fori_loop(0, RPT // ZROWS, zcopy, 0)
            plsc.subcore_barrier()

            def block(ib, carry):
                row0 = q * NCG + coff + ib * BCH
                pltpu.sync_copy(dst_hbm.at[pl.ds(row0, BCH)], dstv)

                def grp(i, carry2):
                    for b in range(GRP):
                        pltpu.async_copy(ones, acc.at[dstv.at[i * GRP + b]],
                                         sem, add=True)
                    for b in range(GRP):
                        pltpu.make_async_copy(
                            ones, acc.at[dstv.at[i * GRP + b]], sem).wait()
                    return carry2
                lax.fori_loop(0, BCH // GRP, grp, 0)
                return carry
            lax.fori_loop(0, nblk, block, 0)
            plsc.subcore_barrier()

            base = (q * 2 + cid) * N_PAD + sid * RPT
            pltpu.sync_copy(acc.at[pl.ds(sid * RPT, RPT)],
                            out_hbm.at[pl.ds(base, RPT)])
            plsc.subcore_barrier()

    return deg


_agg_deg = _make_deg()
_agg_f1 = _make_agg(F1, 2)
_agg_f2 = _make_agg(F2, 4)
_agg_f3 = _make_agg(F3, 4)


def _prep1(deg_parts, x, W1):
    def body(deg_ref, x_ref, w_ref, g_ref, dinv_ref):
        degs = deg_ref[0, 0] + deg_ref[0, 1]
        deg = degs[:, 0:1] + 1.0
        dinv = lax.rsqrt(jnp.maximum(deg, 1.0))
        h = jnp.dot(x_ref[0], w_ref[...], preferred_element_type=jnp.float32,
                    precision=_HI)
        g_ref[0] = h * dinv
        dinv_ref[0] = dinv

    return pl.pallas_call(
        body,
        grid=(2, GB),
        in_specs=[
            pl.BlockSpec((1, 2, BLK, DEG_W), lambda q, i: (q, 0, i, 0)),
            pl.BlockSpec((1, BLK, D), lambda q, i: (q, i, 0)),
            pl.BlockSpec((D, F1), lambda q, i: (0, 0)),
        ],
        out_specs=[
            pl.BlockSpec((1, BLK, F1), lambda q, i: (q, i, 0)),
            pl.BlockSpec((1, BLK, 1), lambda q, i: (q, i, 0)),
        ],
        out_shape=[jax.ShapeDtypeStruct((2, N_PAD, F1), jnp.float32),
                   jax.ShapeDtypeStruct((2, N_PAD, 1), jnp.float32)],
    )(deg_parts, x, W1)


def _prep_mid(S, g, dinv, b2d, W, Fin, Fout):
    def body(s_ref, g_ref, d_ref, b_ref, w_ref, o_ref):
        ssum = s_ref[0, 0] + s_ref[0, 1]
        xn = jnp.maximum(d_ref[0] * (ssum + g_ref[0]) + b_ref[...], 0.0)
        o_ref[0] = jnp.dot(xn, w_ref[...], preferred_element_type=jnp.float32,
                           precision=_HI) * d_ref[0]

    return pl.pallas_call(
        body,
        grid=(2, GB),
        in_specs=[
            pl.BlockSpec((1, 2, BLK, Fin), lambda q, i: (q, 0, i, 0)),
            pl.BlockSpec((1, BLK, Fin), lambda q, i: (q, i, 0)),
            pl.BlockSpec((1, BLK, 1), lambda q, i: (q, i, 0)),
            pl.BlockSpec((1, Fin), lambda q, i: (0, 0)),
            pl.BlockSpec((Fin, Fout), lambda q, i: (0, 0)),
        ],
        out_specs=pl.BlockSpec((1, BLK, Fout), lambda q, i: (q, i, 0)),
        out_shape=jax.ShapeDtypeStruct((2, N_PAD, Fout), jnp.float32),
    )(S, g, dinv, b2d, W)


def _final(S3, g3, dinv, b3_2d, W_att, W_ntn, W_blockT, b_ntnT):
    def body(s_ref, g_ref, d_ref, b_ref, watt_ref, wntn_ref, wblkT_ref,
             bntn_ref, o_ref):
        rows = lax.broadcasted_iota(jnp.int32, (N_PAD, 1), 0)
        valid = rows < N
        ps = []
        for q in range(2):
            a = d_ref[q] * (s_ref[q, 0] + s_ref[q, 1] + g_ref[q]) + b_ref[...]
            a = jnp.where(valid, a, 0.0)
            m = jnp.sum(a, axis=0, keepdims=True) / N
            ctx = jnp.tanh(jnp.dot(m, watt_ref[...],
                                   preferred_element_type=jnp.float32,
                                   precision=_HI))
            logits = jnp.sum(a * ctx, axis=1, keepdims=True)
            s = jax.nn.sigmoid(logits)
            s = jnp.where(valid, s, 0.0)
            ps.append(jnp.sum(a * s, axis=0, keepdims=True))
        p1, p2 = ps

        sc = jnp.zeros((1, K), jnp.float32)
        for i in range(F3):
            row = jnp.dot(p2, wntn_ref[i], preferred_element_type=jnp.float32,
                          precision=_HI)
            sc = sc + p1[:, i:i + 1] * row
        comb = jnp.concatenate([p1, p2], axis=1)
        blk = jnp.dot(comb, wblkT_ref[...],
                      preferred_element_type=jnp.float32, precision=_HI)
        o_ref[...] = jnp.maximum(sc + blk + bntn_ref[...], 0.0)

    return pl.pallas_call(
        body,
        out_shape=jax.ShapeDtypeStruct((1, K), jnp.float32),
    )(S3, g3, dinv, b3_2d, W_att, W_ntn, W_blockT, b_ntnT)


def _pack_edges(ei, off):
    src = jnp.concatenate(
        [ei[0].astype(jnp.int32) + off,
         jnp.full((EP - E,), off + DUMMY, jnp.int32)])
    dst = jnp.concatenate(
        [ei[1].astype(jnp.int32),
         jnp.full((EP - E,), DUMMY, jnp.int32)])
    return src.reshape(NCG, CW), dst.reshape(NCG, CW)


def kernel(features_1, edge_index_1, features_2, edge_index_2, index,
           W1, b1, W2, b2, W3, b3, W_att, W_ntn, W_block, b_ntn):
    x = jnp.stack([jnp.pad(features_1, ((0, N_PAD - N), (0, 0))),
                   jnp.pad(features_2, ((0, N_PAD - N), (0, 0)))])

    s1, t1 = _pack_edges(edge_index_1, 0)
    s2, t2 = _pack_edges(edge_index_2, N_PAD)
    src_all = jnp.concatenate([s1, s2], axis=0)
    dst_all = jnp.concatenate([t1, t2], axis=0)

    deg_parts = _agg_deg(dst_all).reshape(2, 2, N_PAD, DEG_W)

    g1, dinv = _prep1(deg_parts, x, W1)
    S1 = _agg_f1(g1.reshape(2 * N_PAD, F1), src_all, dst_all
                 ).reshape(2, 2, N_PAD, F1)
    g2 = _prep_mid(S1, g1, dinv, b1.reshape(1, F1), W2, F1, F2)
    S2 = _agg_f2(g2.reshape(2 * N_PAD, F2), src_all, dst_all
                 ).reshape(2, 2, N_PAD, F2)
    g3 = _prep_mid(S2, g2, dinv, b2.reshape(1, F2), W3, F2, F3)
    S3 = _agg_f3(g3.reshape(2 * N_PAD, F3), src_all, dst_all
                 ).reshape(2, 2, N_PAD, F3)

    return _final(S3, g3, dinv, b3.reshape(1, F3), W_att, W_ntn,
                  W_block.T, b_ntn.reshape(1, K))


# speedup vs baseline: 28.1106x; 2.4743x over previous
import functools

import jax
import jax.numpy as jnp
from jax import lax
from jax.experimental import pallas as pl
from jax.experimental.pallas import tpu as pltpu
from jax.experimental.pallas import tpu_sc as plsc

N = 10000
E = 320000
D = 128
F1, F2, F3, K = 128, 64, 32, 16

NW = 32
NCHUNK = 80
CW = 128
EP = NW * NCHUNK * CW
NCG = 2 * 16 * NCHUNK
NC0 = 80
NC1 = 160 - NC0
BCH = 40
N_PAD = 10112
DUMMY = N
RPT = N_PAD // 16
ZROWS = 8
DEG_W = 16

BLK = 1264
GB = N_PAD // BLK

_HI = lax.Precision.HIGHEST


def _make_agg(F, nbuf):
    mesh = plsc.VectorSubcoreMesh(core_axis_name="c", subcore_axis_name="s")

    @functools.partial(
        pl.kernel,
        mesh=mesh,
        compiler_params=pltpu.CompilerParams(use_tc_tiling_on_sc=False),
        out_type=jax.ShapeDtypeStruct((4 * N_PAD, F), jnp.float32),
        scratch_types=[
            pltpu.VMEM((BCH, CW), jnp.int32),
            pltpu.VMEM((BCH, CW), jnp.int32),
            [pltpu.VMEM((CW, F), jnp.float32)] * nbuf,
            pltpu.VMEM((ZROWS, F), jnp.float32),
            pltpu.VMEM_SHARED((N_PAD, F), jnp.float32),
            [pltpu.SemaphoreType.DMA] * nbuf,
        ],
    )
    def agg(g_hbm, src_hbm, dst_hbm, out_hbm, srcv, dstv, bufs, zbuf, acc,
            sems):
        cid = lax.axis_index("c")
        sid = lax.axis_index("s")
        coff = sid * (NC0 + NC1) + cid * NC0
        nblk = jnp.where(cid == 0, NC0 // BCH, NC1 // BCH)

        def zfill(r, carry):
            for c in range(F // 16):
                zbuf[r, pl.ds(c * 16, 16)] = jnp.zeros((16,), jnp.float32)
            return carry
        lax.fori_loop(0, ZROWS, zfill, 0)

        for q in range(2):
            def zcopy(c, carry):
                pltpu.sync_copy(zbuf, acc.at[pl.ds(sid * RPT + c * ZROWS, ZROWS)])
                return carry
            lax.fori_loop(0, RPT // ZROWS, zcopy, 0)
            plsc.subcore_barrier()

            def block(ib, carry):
                row0 = q * NCG + coff + ib * BCH
                pltpu.sync_copy(src_hbm.at[pl.ds(row0, BCH)], srcv)
                pltpu.sync_copy(dst_hbm.at[pl.ds(row0, BCH)], dstv)

                for b in range(nbuf):
                    pltpu.async_copy(g_hbm.at[srcv.at[b]], bufs[b], sems[b])

                def grp(i, carry2):
                    for b in range(nbuf):
                        j = i * nbuf + b
                        pltpu.make_async_copy(
                            g_hbm.at[srcv.at[j]], bufs[b], sems[b]).wait()
                        pltpu.sync_copy(bufs[b], acc.at[dstv.at[j]], add=True)

                        @pl.when(j + nbuf < BCH)
                        def _fire():
                            pltpu.async_copy(
                                g_hbm.at[srcv.at[j + nbuf]], bufs[b], sems[b])
                    return carry2
                lax.fori_loop(0, BCH // nbuf, grp, 0)
                return carry
            lax.fori_loop(0, nblk, block, 0)
            plsc.subcore_barrier()

            base = (q * 2 + cid) * N_PAD + sid * RPT
            pltpu.sync_copy(acc.at[pl.ds(sid * RPT, RPT)],
                            out_hbm.at[pl.ds(base, RPT)])
            plsc.subcore_barrier()

    return agg


def _make_deg():
    mesh = plsc.VectorSubcoreMesh(core_axis_name="c", subcore_axis_name="s")
    GRP = 10

    @functools.partial(
        pl.kernel,
        mesh=mesh,
        compiler_params=pltpu.CompilerParams(use_tc_tiling_on_sc=False),
        out_type=jax.ShapeDtypeStruct((4 * N_PAD, DEG_W), jnp.float32),
        scratch_types=[
            pltpu.VMEM((BCH, CW), jnp.int32),
            pltpu.VMEM((CW, DEG_W), jnp.float32),
            pltpu.VMEM((ZROWS, DEG_W), jnp.float32),
            pltpu.VMEM_SHARED((N_PAD, DEG_W), jnp.float32),
            pltpu.SemaphoreType.DMA,
        ],
    )
    def deg(dst_hbm, out_hbm, dstv, ones, zbuf, acc, sem):
        cid = lax.axis_index("c")
        sid = lax.axis_index("s")
        coff = sid * (NC0 + NC1) + cid * NC0
        nblk = jnp.where(cid == 0, NC0 // BCH, NC1 // BCH)

        def fill(r, carry):
            zbuf[r, pl.ds(0, 16)] = jnp.zeros((16,), jnp.float32)
            return carry
        lax.fori_loop(0, ZROWS, fill, 0)

        def ofill(r, carry):
            ones[r, pl.ds(0, 16)] = jnp.ones((16,), jnp.float32)
            return carry
        lax.fori_loop(0, CW, ofill, 0)

        for q in range(2):
            def zcopy(c, carry):
                pltpu.sync_copy(zbuf, acc.at[pl.ds(sid * RPT + c * ZROWS, ZROWS)])
                return carry
            lax.fori_loop(0, RPT // ZROWS, zcopy, 0)
            plsc.subcore_barrier()

            def block(ib, carry):
                row0 = q * NCG + coff + ib * BCH
                pltpu.sync_copy(dst_hbm.at[pl.ds(row0, BCH)], dstv)

                def grp(i, carry2):
                    for b in range(GRP):
                        pltpu.async_copy(ones, acc.at[dstv.at[i * GRP + b]],
                                         sem, add=True)
                    for b in range(GRP):
                        pltpu.make_async_copy(
                            ones, acc.at[dstv.at[i * GRP + b]], sem).wait()
                    return carry2
                lax.fori_loop(0, BCH // GRP, grp, 0)
                return carry
            lax.fori_loop(0, nblk, block, 0)
            plsc.subcore_barrier()

            base = (q * 2 + cid) * N_PAD + sid * RPT
            pltpu.sync_copy(acc.at[pl.ds(sid * RPT, RPT)],
                            out_hbm.at[pl.ds(base, RPT)])
            plsc.subcore_barrier()

    return deg


_agg_deg = _make_deg()
_agg_f1 = _make_agg(F1, 2)
_agg_f2 = _make_agg(F2, 4)
_agg_f3 = _make_agg(F3, 4)


def _prep1(deg_parts, x, W1):
    def body(deg_ref, x_ref, w_ref, g_ref, dinv_ref):
        degs = deg_ref[0, 0] + deg_ref[0, 1]
        deg = degs[:, 0:1] + 1.0
        dinv = lax.rsqrt(jnp.maximum(deg, 1.0))
        h = jnp.dot(x_ref[0], w_ref[...], preferred_element_type=jnp.float32,
                    precision=_HI)
        g_ref[0] = h * dinv
        dinv_ref[0] = dinv

    return pl.pallas_call(
        body,
        grid=(2, GB),
        in_specs=[
            pl.BlockSpec((1, 2, BLK, DEG_W), lambda q, i: (q, 0, i, 0)),
            pl.BlockSpec((1, BLK, D), lambda q, i: (q, i, 0)),
            pl.BlockSpec((D, F1), lambda q, i: (0, 0)),
        ],
        out_specs=[
            pl.BlockSpec((1, BLK, F1), lambda q, i: (q, i, 0)),
            pl.BlockSpec((1, BLK, 1), lambda q, i: (q, i, 0)),
        ],
        out_shape=[jax.ShapeDtypeStruct((2, N_PAD, F1), jnp.float32),
                   jax.ShapeDtypeStruct((2, N_PAD, 1), jnp.float32)],
    )(deg_parts, x, W1)


def _prep_mid(S, g, dinv, b2d, W, Fin, Fout):
    def body(s_ref, g_ref, d_ref, b_ref, w_ref, o_ref):
        ssum = s_ref[0, 0] + s_ref[0, 1]
        xn = jnp.maximum(d_ref[0] * (ssum + g_ref[0]) + b_ref[...], 0.0)
        o_ref[0] = jnp.dot(xn, w_ref[...], preferred_element_type=jnp.float32,
                           precision=_HI) * d_ref[0]

    return pl.pallas_call(
        body,
        grid=(2, GB),
        in_specs=[
            pl.BlockSpec((1, 2, BLK, Fin), lambda q, i: (q, 0, i, 0)),
            pl.BlockSpec((1, BLK, Fin), lambda q, i: (q, i, 0)),
            pl.BlockSpec((1, BLK, 1), lambda q, i: (q, i, 0)),
            pl.BlockSpec((1, Fin), lambda q, i: (0, 0)),
            pl.BlockSpec((Fin, Fout), lambda q, i: (0, 0)),
        ],
        out_specs=pl.BlockSpec((1, BLK, Fout), lambda q, i: (q, i, 0)),
        out_shape=jax.ShapeDtypeStruct((2, N_PAD, Fout), jnp.float32),
    )(S, g, dinv, b2d, W)


def _final(S3, g3, dinv, b3_2d, W_att, W_ntn, W_blockT, b_ntnT):
    def body(s_ref, g_ref, d_ref, b_ref, watt_ref, wntn_ref, wblkT_ref,
             bntn_ref, o_ref):
        rows = lax.broadcasted_iota(jnp.int32, (N_PAD, 1), 0)
        valid = rows < N
        ps = []
        for q in range(2):
            a = d_ref[q] * (s_ref[q, 0] + s_ref[q, 1] + g_ref[q]) + b_ref[...]
            a = jnp.where(valid, a, 0.0)
            m = jnp.sum(a, axis=0, keepdims=True) / N
            ctx = jnp.tanh(jnp.dot(m, watt_ref[...],
                                   preferred_element_type=jnp.float32,
                                   precision=_HI))
            logits = jnp.sum(a * ctx, axis=1, keepdims=True)
            s = jax.nn.sigmoid(logits)
            s = jnp.where(valid, s, 0.0)
            ps.append(jnp.sum(a * s, axis=0, keepdims=True))
        p1, p2 = ps

        sc = jnp.zeros((1, K), jnp.float32)
        for i in range(F3):
            row = jnp.dot(p2, wntn_ref[i], preferred_element_type=jnp.float32,
                          precision=_HI)
            sc = sc + p1[:, i:i + 1] * row
        comb = jnp.concatenate([p1, p2], axis=1)
        blk = jnp.dot(comb, wblkT_ref[...],
                      preferred_element_type=jnp.float32, precision=_HI)
        o_ref[...] = jnp.maximum(sc + blk + bntn_ref[...], 0.0)

    return pl.pallas_call(
        body,
        out_shape=jax.ShapeDtypeStruct((1, K), jnp.float32),
    )(S3, g3, dinv, b3_2d, W_att, W_ntn, W_blockT, b_ntnT)


def _pack_edges(ei, off):
    spread = DUMMY + (jnp.arange(EP - E, dtype=jnp.int32) % (N_PAD - N))
    src = jnp.concatenate([ei[0].astype(jnp.int32) + off, off + spread])
    dst = jnp.concatenate([ei[1].astype(jnp.int32), spread])
    return src.reshape(NCG, CW), dst.reshape(NCG, CW)


def kernel(features_1, edge_index_1, features_2, edge_index_2, index,
           W1, b1, W2, b2, W3, b3, W_att, W_ntn, W_block, b_ntn):
    x = jnp.stack([jnp.pad(features_1, ((0, N_PAD - N), (0, 0))),
                   jnp.pad(features_2, ((0, N_PAD - N), (0, 0)))])

    s1, t1 = _pack_edges(edge_index_1, 0)
    s2, t2 = _pack_edges(edge_index_2, N_PAD)
    src_all = jnp.concatenate([s1, s2], axis=0)
    dst_all = jnp.concatenate([t1, t2], axis=0)

    deg_parts = _agg_deg(dst_all).reshape(2, 2, N_PAD, DEG_W)

    g1, dinv = _prep1(deg_parts, x, W1)
    S1 = _agg_f1(g1.reshape(2 * N_PAD, F1), src_all, dst_all
                 ).reshape(2, 2, N_PAD, F1)
    g2 = _prep_mid(S1, g1, dinv, b1.reshape(1, F1), W2, F1, F2)
    S2 = _agg_f2(g2.reshape(2 * N_PAD, F2), src_all, dst_all
                 ).reshape(2, 2, N_PAD, F2)
    g3 = _prep_mid(S2, g2, dinv, b2.reshape(1, F2), W3, F2, F3)
    S3 = _agg_f3(g3.reshape(2 * N_PAD, F3), src_all, dst_all
                 ).reshape(2, 2, N_PAD, F3)

    return _final(S3, g3, dinv, b3.reshape(1, F3), W_att, W_ntn,
                  W_block.T, b_ntn.reshape(1, K))


# speedup vs baseline: 28.1244x; 1.0005x over previous
import functools

import jax
import jax.numpy as jnp
from jax import lax
from jax.experimental import pallas as pl
from jax.experimental.pallas import tpu as pltpu
from jax.experimental.pallas import tpu_sc as plsc

N = 10000
E = 320000
D = 128
F1, F2, F3, K = 128, 64, 32, 16

NW = 32
NCHUNK = 80
CW = 128
EP = NW * NCHUNK * CW
NCG = 2 * 16 * NCHUNK
NC0 = 80
NC1 = 160 - NC0
BCH = 40
N_PAD = 10112
DUMMY = N
RPT = N_PAD // 16
ZROWS = 8
DEG_W = 16

BLK = 1264
GB = N_PAD // BLK

_HI = lax.Precision.HIGHEST


def _make_agg(F, nbuf):
    mesh = plsc.VectorSubcoreMesh(core_axis_name="c", subcore_axis_name="s")

    @functools.partial(
        pl.kernel,
        mesh=mesh,
        compiler_params=pltpu.CompilerParams(use_tc_tiling_on_sc=False),
        out_type=jax.ShapeDtypeStruct((4 * N_PAD, F), jnp.float32),
        scratch_types=[
            pltpu.VMEM((BCH, CW), jnp.int32),
            pltpu.VMEM((BCH, CW), jnp.int32),
            [pltpu.VMEM((CW, F), jnp.float32)] * nbuf,
            pltpu.VMEM((ZROWS, F), jnp.float32),
            pltpu.VMEM_SHARED((N_PAD, F), jnp.float32),
            [pltpu.SemaphoreType.DMA] * nbuf,
        ],
    )
    def agg(g_hbm, src_hbm, dst_hbm, out_hbm, srcv, dstv, bufs, zbuf, acc,
            sems):
        cid = lax.axis_index("c")
        sid = lax.axis_index("s")
        coff = sid * (NC0 + NC1) + cid * NC0
        nblk = jnp.where(cid == 0, NC0 // BCH, NC1 // BCH)

        def zfill(r, carry):
            for c in range(F // 16):
                zbuf[r, pl.ds(c * 16, 16)] = jnp.zeros((16,), jnp.float32)
            return carry
        lax.fori_loop(0, ZROWS, zfill, 0)

        for q in range(2):
            def zcopy(c, carry):
                pltpu.sync_copy(zbuf, acc.at[pl.ds(sid * RPT + c * ZROWS, ZROWS)])
                return carry
            lax.fori_loop(0, RPT // ZROWS, zcopy, 0)
            plsc.subcore_barrier()

            def block(ib, carry):
                row0 = q * NCG + coff + ib * BCH
                pltpu.sync_copy(src_hbm.at[pl.ds(row0, BCH)], srcv)
                pltpu.sync_copy(dst_hbm.at[pl.ds(row0, BCH)], dstv)

                for b in range(nbuf):
                    pltpu.async_copy(g_hbm.at[srcv.at[b]], bufs[b], sems[b])

                def grp(i, carry2):
                    for b in range(nbuf):
                        j = i * nbuf + b
                        pltpu.make_async_copy(
                            g_hbm.at[srcv.at[j]], bufs[b], sems[b]).wait()
                        pltpu.sync_copy(bufs[b], acc.at[dstv.at[j]], add=True)

                        @pl.when(j + nbuf < BCH)
                        def _fire():
                            pltpu.async_copy(
                                g_hbm.at[srcv.at[j + nbuf]], bufs[b], sems[b])
                    return carry2
                lax.fori_loop(0, BCH // nbuf, grp, 0)
                return carry
            lax.fori_loop(0, nblk, block, 0)
            plsc.subcore_barrier()

            base = (q * 2 + cid) * N_PAD + sid * RPT
            pltpu.sync_copy(acc.at[pl.ds(sid * RPT, RPT)],
                            out_hbm.at[pl.ds(base, RPT)])
            plsc.subcore_barrier()

    return agg


def _make_deg():
    mesh = plsc.VectorSubcoreMesh(core_axis_name="c", subcore_axis_name="s")
    GRP = 10

    @functools.partial(
        pl.kernel,
        mesh=mesh,
        compiler_params=pltpu.CompilerParams(use_tc_tiling_on_sc=False),
        out_type=jax.ShapeDtypeStruct((4 * N_PAD, DEG_W), jnp.float32),
        scratch_types=[
            pltpu.VMEM((BCH, CW), jnp.int32),
            pltpu.VMEM((CW, DEG_W), jnp.float32),
            pltpu.VMEM((ZROWS, DEG_W), jnp.float32),
            pltpu.VMEM_SHARED((N_PAD, DEG_W), jnp.float32),
            pltpu.SemaphoreType.DMA,
        ],
    )
    def deg(dst_hbm, out_hbm, dstv, ones, zbuf, acc, sem):
        cid = lax.axis_index("c")
        sid = lax.axis_index("s")
        coff = sid * (NC0 + NC1) + cid * NC0
        nblk = jnp.where(cid == 0, NC0 // BCH, NC1 // BCH)

        def fill(r, carry):
            zbuf[r, pl.ds(0, 16)] = jnp.zeros((16,), jnp.float32)
            return carry
        lax.fori_loop(0, ZROWS, fill, 0)

        def ofill(r, carry):
            ones[r, pl.ds(0, 16)] = jnp.ones((16,), jnp.float32)
            return carry
        lax.fori_loop(0, CW, ofill, 0)

        for q in range(2):
            def zcopy(c, carry):
                pltpu.sync_copy(zbuf, acc.at[pl.ds(sid * RPT + c * ZROWS, ZROWS)])
                return carry
            lax.fori_loop(0, RPT // ZROWS, zcopy, 0)
            plsc.subcore_barrier()

            def block(ib, carry):
                row0 = q * NCG + coff + ib * BCH
                pltpu.sync_copy(dst_hbm.at[pl.ds(row0, BCH)], dstv)

                def grp(i, carry2):
                    for b in range(GRP):
                        pltpu.async_copy(ones, acc.at[dstv.at[i * GRP + b]],
                                         sem, add=True)
                    for b in range(GRP):
                        pltpu.make_async_copy(
                            ones, acc.at[dstv.at[i * GRP + b]], sem).wait()
                    return carry2
                lax.fori_loop(0, BCH // GRP, grp, 0)
                return carry
            lax.fori_loop(0, nblk, block, 0)
            plsc.subcore_barrier()

            base = (q * 2 + cid) * N_PAD + sid * RPT
            pltpu.sync_copy(acc.at[pl.ds(sid * RPT, RPT)],
                            out_hbm.at[pl.ds(base, RPT)])
            plsc.subcore_barrier()

    return deg


_agg_deg = _make_deg()
_agg_f1 = _make_agg(F1, 2)
_agg_f2 = _make_agg(F2, 4)
_agg_f3 = _make_agg(F3, 4)


def _prep1(deg_parts, x, W1):
    def body(dega_ref, degb_ref, x_ref, w_ref, g_ref, dinv_ref):
        degs = dega_ref[...] + degb_ref[...]
        deg = degs[:, 0:1] + 1.0
        dinv = lax.rsqrt(jnp.maximum(deg, 1.0))
        h = jnp.dot(x_ref[...], w_ref[...], preferred_element_type=jnp.float32,
                    precision=_HI)
        g_ref[...] = h * dinv
        dinv_ref[...] = dinv

    return pl.pallas_call(
        body,
        grid=(2, GB),
        in_specs=[
            pl.BlockSpec((BLK, DEG_W), lambda q, i: (2 * GB * q + i, 0)),
            pl.BlockSpec((BLK, DEG_W), lambda q, i: (2 * GB * q + GB + i, 0)),
            pl.BlockSpec((BLK, D), lambda q, i: (GB * q + i, 0)),
            pl.BlockSpec((D, F1), lambda q, i: (0, 0)),
        ],
        out_specs=[
            pl.BlockSpec((BLK, F1), lambda q, i: (GB * q + i, 0)),
            pl.BlockSpec((BLK, 1), lambda q, i: (GB * q + i, 0)),
        ],
        out_shape=[jax.ShapeDtypeStruct((2 * N_PAD, F1), jnp.float32),
                   jax.ShapeDtypeStruct((2 * N_PAD, 1), jnp.float32)],
    )(deg_parts, deg_parts, x, W1)


def _prep_mid(S, g, dinv, b2d, W, Fin, Fout):
    def body(sa_ref, sb_ref, g_ref, d_ref, b_ref, w_ref, o_ref):
        ssum = sa_ref[...] + sb_ref[...]
        xn = jnp.maximum(d_ref[...] * (ssum + g_ref[...]) + b_ref[...], 0.0)
        o_ref[...] = jnp.dot(xn, w_ref[...], preferred_element_type=jnp.float32,
                             precision=_HI) * d_ref[...]

    return pl.pallas_call(
        body,
        grid=(2, GB),
        in_specs=[
            pl.BlockSpec((BLK, Fin), lambda q, i: (2 * GB * q + i, 0)),
            pl.BlockSpec((BLK, Fin), lambda q, i: (2 * GB * q + GB + i, 0)),
            pl.BlockSpec((BLK, Fin), lambda q, i: (GB * q + i, 0)),
            pl.BlockSpec((BLK, 1), lambda q, i: (GB * q + i, 0)),
            pl.BlockSpec((1, Fin), lambda q, i: (0, 0)),
            pl.BlockSpec((Fin, Fout), lambda q, i: (0, 0)),
        ],
        out_specs=pl.BlockSpec((BLK, Fout), lambda q, i: (GB * q + i, 0)),
        out_shape=jax.ShapeDtypeStruct((2 * N_PAD, Fout), jnp.float32),
    )(S, S, g, dinv, b2d, W)


def _final(S3, g3, dinv, b3_2d, W_att, W_ntn, W_blockT, b_ntnT):
    def body(s_ref, g_ref, d_ref, b_ref, watt_ref, wntn_ref, wblkT_ref,
             bntn_ref, o_ref):
        rows = lax.broadcasted_iota(jnp.int32, (N_PAD, 1), 0)
        valid = rows < N
        ps = []
        for q in range(2):
            s0 = s_ref[pl.ds(2 * q * N_PAD, N_PAD), :]
            s1 = s_ref[pl.ds((2 * q + 1) * N_PAD, N_PAD), :]
            gq = g_ref[pl.ds(q * N_PAD, N_PAD), :]
            dq = d_ref[pl.ds(q * N_PAD, N_PAD), :]
            a = dq * (s0 + s1 + gq) + b_ref[...]
            a = jnp.where(valid, a, 0.0)
            m = jnp.sum(a, axis=0, keepdims=True) / N
            ctx = jnp.tanh(jnp.dot(m, watt_ref[...],
                                   preferred_element_type=jnp.float32,
                                   precision=_HI))
            logits = jnp.sum(a * ctx, axis=1, keepdims=True)
            s = jax.nn.sigmoid(logits)
            s = jnp.where(valid, s, 0.0)
            ps.append(jnp.sum(a * s, axis=0, keepdims=True))
        p1, p2 = ps

        sc = jnp.zeros((1, K), jnp.float32)
        for i in range(F3):
            row = jnp.dot(p2, wntn_ref[i], preferred_element_type=jnp.float32,
                          precision=_HI)
            sc = sc + p1[:, i:i + 1] * row
        comb = jnp.concatenate([p1, p2], axis=1)
        blk = jnp.dot(comb, wblkT_ref[...],
                      preferred_element_type=jnp.float32, precision=_HI)
        o_ref[...] = jnp.maximum(sc + blk + bntn_ref[...], 0.0)

    return pl.pallas_call(
        body,
        out_shape=jax.ShapeDtypeStruct((1, K), jnp.float32),
    )(S3, g3, dinv, b3_2d, W_att, W_ntn, W_blockT, b_ntnT)


def _pack_edges(ei, off):
    spread = DUMMY + (jnp.arange(EP - E, dtype=jnp.int32) % (N_PAD - N))
    src = jnp.concatenate([ei[0].astype(jnp.int32) + off, off + spread])
    dst = jnp.concatenate([ei[1].astype(jnp.int32), spread])
    return src.reshape(NCG, CW), dst.reshape(NCG, CW)


def kernel(features_1, edge_index_1, features_2, edge_index_2, index,
           W1, b1, W2, b2, W3, b3, W_att, W_ntn, W_block, b_ntn):
    x = jnp.concatenate([jnp.pad(features_1, ((0, N_PAD - N), (0, 0))),
                         jnp.pad(features_2, ((0, N_PAD - N), (0, 0)))])

    s1, t1 = _pack_edges(edge_index_1, 0)
    s2, t2 = _pack_edges(edge_index_2, N_PAD)
    src_all = jnp.concatenate([s1, s2], axis=0)
    dst_all = jnp.concatenate([t1, t2], axis=0)

    deg_parts = _agg_deg(dst_all)

    g1, dinv = _prep1(deg_parts, x, W1)
    S1 = _agg_f1(g1, src_all, dst_all)
    g2 = _prep_mid(S1, g1, dinv, b1.reshape(1, F1), W2, F1, F2)
    S2 = _agg_f2(g2, src_all, dst_all)
    g3 = _prep_mid(S2, g2, dinv, b2.reshape(1, F2), W3, F2, F3)
    S3 = _agg_f3(g3, src_all, dst_all)

    return _final(S3, g3, dinv, b3.reshape(1, F3), W_att, W_ntn,
                  W_block.T, b_ntn.reshape(1, K))
